# baseline (speedup 1.0000x reference)
import dataclasses
import functools

import jax
import jax.numpy as jnp
from jax import lax
from jax.experimental import pallas as pl
from jax.experimental.pallas import tpu as pltpu
from jax.experimental.pallas import tpu_sc as plsc

N_NODES = 10000
N_EDGES = 160000
D_IN = 256
D_HID = 256
D = D_IN + D_HID

BM = 1000
BE = 1280

C_A = 40
NCH_A = N_EDGES // 32 // C_A
C_B = 40
NCH_B = N_EDGES // 16 // C_B

_mesh = plsc.VectorSubcoreMesh(core_axis_name="c", subcore_axis_name="s")

_sc_params = pltpu.CompilerParams()
for _f, _v in (("needs_layout_passes", False), ("use_tc_tiling_on_sc", False)):
    if _f in pltpu.CompilerParams.__dataclass_fields__:
        _sc_params = dataclasses.replace(_sc_params, **{_f: _v})


def _embed_body(x_ref, w_ref, b_ref, o_ref):
    h = jnp.tanh(jnp.dot(x_ref[...], w_ref[...],
                         preferred_element_type=jnp.float32) + b_ref[...])
    o_ref[:, :D_IN] = h
    o_ref[:, D_IN:] = x_ref[...]


def _embed(X2, W_in, b_in):
    return pl.pallas_call(
        _embed_body,
        grid=(N_NODES // BM,),
        in_specs=[
            pl.BlockSpec((BM, D_IN), lambda i: (i, 0)),
            pl.BlockSpec((D_IN, D_HID), lambda i: (0, 0)),
            pl.BlockSpec((1, D_HID), lambda i: (0, 0)),
        ],
        out_specs=pl.BlockSpec((BM, D), lambda i: (i, 0)),
        out_shape=jax.ShapeDtypeStruct((N_NODES, D), jnp.float32),
    )(X2, W_in, b_in)


def _proj_body(h_ref, w_ref, o_ref):
    o_ref[...] = jnp.dot(h_ref[...], w_ref[...],
                         preferred_element_type=jnp.float32)


def _proj(H, W):
    n = W.shape[1]
    return pl.pallas_call(
        _proj_body,
        grid=(N_NODES // BM,),
        in_specs=[
            pl.BlockSpec((BM, D), lambda i: (i, 0)),
            pl.BlockSpec((D, n), lambda i: (0, 0)),
        ],
        out_specs=pl.BlockSpec((BM, n), lambda i: (i, 0)),
        out_shape=jax.ShapeDtypeStruct((N_NODES, n), jnp.float32),
    )(H, W)


def _proj_split_body(h_ref, w_ref, o_ref):
    o_ref[0] = jnp.dot(h_ref[...], w_ref[...],
                       preferred_element_type=jnp.float32)


def _proj_split(H, W):
    return pl.pallas_call(
        _proj_split_body,
        grid=(N_NODES // BM, 2),
        in_specs=[
            pl.BlockSpec((BM, D), lambda i, j: (i, 0)),
            pl.BlockSpec((D, 128), lambda i, j: (0, j)),
        ],
        out_specs=pl.BlockSpec((1, BM, 128), lambda i, j: (j, i, 0)),
        out_shape=jax.ShapeDtypeStruct((2, N_NODES, 128), jnp.float32),
    )(H, W)


def _escore_body(z_ref, be1_ref, w2_ref, be2_ref, o_ref):
    t = jnp.tanh(z_ref[...] + be1_ref[...])
    s = jnp.dot(t, w2_ref[...], preferred_element_type=jnp.float32)
    e = jax.nn.sigmoid(s + be2_ref[0, 0])
    o_ref[pl.ds(pl.program_id(0) * BE, BE)] = e[:, 0]


def _escore(Z, be1, We2, be2):
    return pl.pallas_call(
        _escore_body,
        grid=(N_EDGES // BE,),
        in_specs=[
            pl.BlockSpec((BE, D_HID), lambda i: (i, 0)),
            pl.BlockSpec((1, D_HID), lambda i: (0, 0)),
            pl.BlockSpec((D_HID, 1), lambda i: (0, 0)),
            pl.BlockSpec((1, 1), lambda i: (0, 0)),
        ],
        out_specs=pl.BlockSpec((N_EDGES,), lambda i: (0,)),
        out_shape=jax.ShapeDtypeStruct((N_EDGES,), jnp.float32),
    )(Z, be1, We2, be2)


def _update_body(s_ref, h_ref, x_ref, wc_ref, bn1_ref, wn2_ref, bn2_ref, o_ref):
    s = jnp.concatenate([s_ref[0], s_ref[1]], axis=-1)
    hmid = jnp.tanh(s + jnp.dot(h_ref[...], wc_ref[...],
                                preferred_element_type=jnp.float32) + bn1_ref[...])
    hn = jnp.tanh(jnp.dot(hmid, wn2_ref[...],
                          preferred_element_type=jnp.float32) + bn2_ref[...])
    o_ref[:, :D_HID] = hn
    o_ref[:, D_HID:] = x_ref[...]


def _update(S, H, X2, Wc, bn1, Wn2, bn2):
    return pl.pallas_call(
        _update_body,
        grid=(N_NODES // BM,),
        in_specs=[
            pl.BlockSpec((2, BM, 128), lambda i: (0, i, 0)),
            pl.BlockSpec((BM, D), lambda i: (i, 0)),
            pl.BlockSpec((BM, D_IN), lambda i: (i, 0)),
            pl.BlockSpec((D, D_HID), lambda i: (0, 0)),
            pl.BlockSpec((1, D_HID), lambda i: (0, 0)),
            pl.BlockSpec((D_HID, D_HID), lambda i: (0, 0)),
            pl.BlockSpec((1, D_HID), lambda i: (0, 0)),
        ],
        out_specs=pl.BlockSpec((BM, D), lambda i: (i, 0)),
        out_shape=jax.ShapeDtypeStruct((N_NODES, D), jnp.float32),
    )(S, H, X2, Wc, bn1, Wn2, bn2)


def _sc_gsum_body(go_hbm, gi_hbm, idxo_hbm, idxi_hbm, z_hbm,
                  idxo_t, idxi_t, ubuf, vbuf, zbuf, gsem0, gsem1, wsem):
    c = lax.axis_index("c")
    s = lax.axis_index("s")
    wid = c * 16 + s
    gsems = (gsem0, gsem1)

    pltpu.sync_copy(idxo_hbm.at[wid], idxo_t)
    pltpu.sync_copy(idxi_hbm.at[wid], idxi_t)

    def issue(ci, b):
        pltpu.make_async_copy(go_hbm.at[idxo_t.at[ci]], ubuf.at[b],
                              gsems[b]).start()
        pltpu.make_async_copy(gi_hbm.at[idxi_t.at[ci]], vbuf.at[b],
                              gsems[b]).start()

    def wait(ci, b):
        pltpu.make_async_copy(go_hbm.at[idxo_t.at[ci]], ubuf.at[b],
                              gsems[b]).wait()
        pltpu.make_async_copy(gi_hbm.at[idxi_t.at[ci]], vbuf.at[b],
                              gsems[b]).wait()

    def zdst(ci):
        return z_hbm.at[pl.ds(wid * (NCH_A * C_A) + ci * C_A, C_A)]

    def work(ci, b):
        ub = ubuf.at[b]
        vb = vbuf.at[b]
        zb = zbuf.at[b]

        @pl.when(ci >= 2)
        def _():
            pltpu.make_async_copy(zbuf.at[b], zdst(ci - 2), wsem).wait()

        @pl.loop(0, C_A)
        def _(j):
            for k in range(16):
                sl = pl.ds(16 * k, 16)
                zb[j, sl] = ub[j, sl] + vb[j, sl]

        pltpu.make_async_copy(zbuf.at[b], zdst(ci), wsem).start()

    issue(0, 0)

    @pl.loop(0, NCH_A - 1, step=2)
    def _(ci):
        for b in range(2):
            cur = ci + b
            issue(cur + 1, 1 - b)
            wait(cur, b)
            work(cur, b)

    wait(NCH_A - 1, 0)
    work(NCH_A - 1, 0)
    pltpu.make_async_copy(zbuf.at[1], zdst(NCH_A - 2), wsem).wait()
    pltpu.make_async_copy(zbuf.at[0], zdst(NCH_A - 1), wsem).wait()


def _sc_gsum(Go, Gi, idxo32, idxi32):
    k = pl.kernel(
        _sc_gsum_body,
        out_type=jax.ShapeDtypeStruct((N_EDGES, D_HID), jnp.float32),
        mesh=_mesh,
        compiler_params=_sc_params,
        scratch_types=[
            pltpu.VMEM((NCH_A, C_A), jnp.int32),
            pltpu.VMEM((NCH_A, C_A), jnp.int32),
            pltpu.VMEM((2, C_A, D_HID), jnp.float32),
            pltpu.VMEM((2, C_A, D_HID), jnp.float32),
            pltpu.VMEM((2, C_A, D_HID), jnp.float32),
            pltpu.SemaphoreType.DMA,
            pltpu.SemaphoreType.DMA,
            pltpu.SemaphoreType.DMA,
        ],
    )
    return k(Go, Gi, idxo32, idxi32)


def _sc_msg_body(pa_hbm, pb_hbm, e_hbm, idxo_hbm, idxi_hbm, s_hbm,
                 acc, idxo_t, idxi_t, ubuf, vbuf, ebuf, m1, m2, zb,
                 gsem0, gsem1, isem0, isem1, isem2, isem3):
    c = lax.axis_index("c")
    s = lax.axis_index("s")
    gsems = (gsem0, gsem1)
    isems = (isem0, isem1, isem2, isem3)

    zv = jnp.zeros((16,), jnp.float32)

    @pl.loop(0, 8)
    def _(r):
        for q in range(8):
            zb[r, pl.ds(16 * q, 16)] = zv

    @pl.when(s < 10)
    def _():
        @pl.loop(0, 125)
        def _(jz):
            pltpu.sync_copy(zb, acc.at[pl.ds(s * 1000 + jz * 8, 8)])

    plsc.subcore_barrier()

    pa = pa_hbm.at[c]
    pb = pb_hbm.at[c]

    def issue_idx(ci, slot):
        pltpu.make_async_copy(idxo_hbm.at[s, ci], idxo_t.at[slot],
                              isems[slot]).start()
        pltpu.make_async_copy(idxi_hbm.at[s, ci], idxi_t.at[slot],
                              isems[slot]).start()

    def wait_idx(ci, slot):
        pltpu.make_async_copy(idxo_hbm.at[s, ci], idxo_t.at[slot],
                              isems[slot]).wait()
        pltpu.make_async_copy(idxi_hbm.at[s, ci], idxi_t.at[slot],
                              isems[slot]).wait()

    def esrc(ci):
        return e_hbm.at[pl.ds(s * (NCH_B * C_B) + ci * C_B, C_B)]

    def issue(ci, slot, b):
        pltpu.make_async_copy(pa.at[idxo_t.at[slot]], ubuf.at[b],
                              gsems[b]).start()
        pltpu.make_async_copy(pb.at[idxi_t.at[slot]], vbuf.at[b],
                              gsems[b]).start()
        pltpu.make_async_copy(esrc(ci), ebuf.at[b], gsems[b]).start()

    def wait(ci, slot, b):
        pltpu.make_async_copy(pa.at[idxo_t.at[slot]], ubuf.at[b],
                              gsems[b]).wait()
        pltpu.make_async_copy(pb.at[idxi_t.at[slot]], vbuf.at[b],
                              gsems[b]).wait()
        pltpu.make_async_copy(esrc(ci), ebuf.at[b], gsems[b]).wait()

    def work(slot, p):
        ub = ubuf.at[p]
        vb = vbuf.at[p]
        eb = ebuf.at[p]

        @pl.loop(0, C_B)
        def _(j):
            ev = plsc.load_gather(eb, [jnp.broadcast_to(j, (16,))])
            for k in range(8):
                sl = pl.ds(16 * k, 16)
                m1[p, j, sl] = ev * ub[j, sl]
                m2[p, j, sl] = ev * vb[j, sl]

        pltpu.sync_copy(m1.at[p], acc.at[idxi_t.at[slot]], add=True)
        pltpu.sync_copy(m2.at[p], acc.at[idxo_t.at[slot]], add=True)

    def body(ci, b):
        cur = ci + b
        p = b % 2
        slot = b % 4

        @pl.when(cur + 1 < NCH_B)
        def _():
            wait_idx(cur + 1, (slot + 1) % 4)
            issue(cur + 1, (slot + 1) % 4, 1 - p)

        wait(cur, slot, p)
        work(slot, p)

        @pl.when(cur + 3 < NCH_B)
        def _():
            issue_idx(cur + 3, (slot + 3) % 4)

    issue_idx(0, 0)
    issue_idx(1, 1)
    issue_idx(2, 2)
    wait_idx(0, 0)
    issue(0, 0, 0)

    @pl.loop(0, NCH_B - 2, step=4)
    def _(ci):
        for b in range(4):
            body(ci, b)

    body(NCH_B - 2, 0)
    body(NCH_B - 2, 1)

    plsc.subcore_barrier()

    @pl.when(s < 10)
    def _():
        pltpu.sync_copy(acc.at[pl.ds(s * 1000, 1000)],
                        s_hbm.at[c, pl.ds(s * 1000, 1000)])


def _sc_msg(TpA, TpB, e, idxo16, idxi16):
    k = pl.kernel(
        _sc_msg_body,
        out_type=jax.ShapeDtypeStruct((2, N_NODES, 128), jnp.float32),
        mesh=_mesh,
        compiler_params=_sc_params,
        scratch_types=[
            pltpu.VMEM_SHARED((N_NODES, 128), jnp.float32),
            pltpu.VMEM((4, C_B), jnp.int32),
            pltpu.VMEM((4, C_B), jnp.int32),
            pltpu.VMEM((2, C_B, 128), jnp.float32),
            pltpu.VMEM((2, C_B, 128), jnp.float32),
            pltpu.VMEM((2, C_B), jnp.float32),
            pltpu.VMEM((2, C_B, 128), jnp.float32),
            pltpu.VMEM((2, C_B, 128), jnp.float32),
            pltpu.VMEM((8, 128), jnp.float32),
            pltpu.SemaphoreType.DMA,
            pltpu.SemaphoreType.DMA,
            pltpu.SemaphoreType.DMA,
            pltpu.SemaphoreType.DMA,
            pltpu.SemaphoreType.DMA,
            pltpu.SemaphoreType.DMA,
        ],
    )
    return k(TpA, TpB, e, idxo16, idxi16)


def kernel(X, edge_in, edge_out, W_in, b_in, We1, be1, We2, be2,
           Wn1, bn1, Wn2, bn2):
    X2 = X[0]
    ii = edge_in[0].astype(jnp.int32)
    oo = edge_out[0].astype(jnp.int32)

    Wc = Wn1[2 * D:]
    b_in2 = b_in.reshape(1, D_HID)
    be1_2 = be1.reshape(1, D_HID)
    be2_2 = be2.reshape(1, 1)
    bn1_2 = bn1.reshape(1, D_HID)
    bn2_2 = bn2.reshape(1, D_HID)

    idxo32 = oo.reshape(32, NCH_A, C_A)
    idxi32 = ii.reshape(32, NCH_A, C_A)
    idxo16 = oo.reshape(16, NCH_B, C_B)
    idxi16 = ii.reshape(16, NCH_B, C_B)

    H = _embed(X2, W_in, b_in2)
    for _ in range(3):
        Go = _proj(H, We1[:D])
        Gi = _proj(H, We1[D:])
        TpA = _proj_split(H, Wn1[:D])
        TpB = _proj_split(H, Wn1[D:2 * D])
        Z = _sc_gsum(Go, Gi, idxo32, idxi32)
        e = _escore(Z, be1_2, We2, be2_2)
        S = _sc_msg(TpA, TpB, e, idxo16, idxi16)
        H = _update(S, H, X2, Wc, bn1_2, Wn2, bn2_2)

    Go = _proj(H, We1[:D])
    Gi = _proj(H, We1[D:])
    Z = _sc_gsum(Go, Gi, idxo32, idxi32)
    e = _escore(Z, be1_2, We2, be2_2)
    return e.reshape(1, N_EDGES)


# speedup vs baseline: 1.2129x; 1.2129x over previous
import dataclasses
import functools

import jax
import jax.numpy as jnp
from jax import lax
from jax.experimental import pallas as pl
from jax.experimental.pallas import tpu as pltpu
from jax.experimental.pallas import tpu_sc as plsc

N_NODES = 10000
N_EDGES = 160000
D_IN = 256
D_HID = 256
D = D_IN + D_HID

BM = 1000
BE = 1280

C_A = 40
NCH_A = N_EDGES // 32 // C_A
C_B = 40
NCH_B = N_EDGES // 16 // C_B

_mesh = plsc.VectorSubcoreMesh(core_axis_name="c", subcore_axis_name="s")

_sc_params = pltpu.CompilerParams()
for _f, _v in (("needs_layout_passes", False), ("use_tc_tiling_on_sc", False)):
    if _f in pltpu.CompilerParams.__dataclass_fields__:
        _sc_params = dataclasses.replace(_sc_params, **{_f: _v})


def _embed_body(x_ref, w_ref, b_ref, o_ref):
    h = jnp.tanh(jnp.dot(x_ref[...], w_ref[...],
                         preferred_element_type=jnp.float32) + b_ref[...])
    o_ref[:, :D_IN] = h
    o_ref[:, D_IN:] = x_ref[...]


def _embed(X2, W_in, b_in):
    return pl.pallas_call(
        _embed_body,
        grid=(N_NODES // BM,),
        in_specs=[
            pl.BlockSpec((BM, D_IN), lambda i: (i, 0)),
            pl.BlockSpec((D_IN, D_HID), lambda i: (0, 0)),
            pl.BlockSpec((1, D_HID), lambda i: (0, 0)),
        ],
        out_specs=pl.BlockSpec((BM, D), lambda i: (i, 0)),
        out_shape=jax.ShapeDtypeStruct((N_NODES, D), jnp.float32),
    )(X2, W_in, b_in)


def _proj_body(h_ref, w_ref, o_ref):
    o_ref[...] = jnp.dot(h_ref[...], w_ref[...],
                         preferred_element_type=jnp.float32)


def _proj(H, W):
    n = W.shape[1]
    return pl.pallas_call(
        _proj_body,
        grid=(N_NODES // BM,),
        in_specs=[
            pl.BlockSpec((BM, D), lambda i: (i, 0)),
            pl.BlockSpec((D, n), lambda i: (0, 0)),
        ],
        out_specs=pl.BlockSpec((BM, n), lambda i: (i, 0)),
        out_shape=jax.ShapeDtypeStruct((N_NODES, n), jnp.float32),
    )(H, W)


def _proj_split_body(h_ref, w_ref, o_ref):
    o_ref[0] = jnp.dot(h_ref[...], w_ref[...],
                       preferred_element_type=jnp.float32)


def _proj_split(H, W):
    return pl.pallas_call(
        _proj_split_body,
        grid=(N_NODES // BM, 2),
        in_specs=[
            pl.BlockSpec((BM, D), lambda i, j: (i, 0)),
            pl.BlockSpec((D, 128), lambda i, j: (0, j)),
        ],
        out_specs=pl.BlockSpec((1, BM, 128), lambda i, j: (j, i, 0)),
        out_shape=jax.ShapeDtypeStruct((2, N_NODES, 128), jnp.float32),
    )(H, W)


def _escore_body(z_ref, be1_ref, w2_ref, be2_ref, o_ref):
    t = jnp.tanh(z_ref[...] + be1_ref[...])
    s = jnp.dot(t, w2_ref[...], preferred_element_type=jnp.float32)
    e = jax.nn.sigmoid(s + be2_ref[0, 0])
    o_ref[pl.ds(pl.program_id(0) * BE, BE)] = e[:, 0]


def _escore(Z, be1, We2, be2):
    return pl.pallas_call(
        _escore_body,
        grid=(N_EDGES // BE,),
        in_specs=[
            pl.BlockSpec((BE, D_HID), lambda i: (i, 0)),
            pl.BlockSpec((1, D_HID), lambda i: (0, 0)),
            pl.BlockSpec((D_HID, 1), lambda i: (0, 0)),
            pl.BlockSpec((1, 1), lambda i: (0, 0)),
        ],
        out_specs=pl.BlockSpec((N_EDGES,), lambda i: (0,)),
        out_shape=jax.ShapeDtypeStruct((N_EDGES,), jnp.float32),
    )(Z, be1, We2, be2)


def _update_body(s_ref, h_ref, x_ref, wc_ref, bn1_ref, wn2_ref, bn2_ref, o_ref):
    s = jnp.concatenate([s_ref[0], s_ref[1]], axis=-1)
    hmid = jnp.tanh(s + jnp.dot(h_ref[...], wc_ref[...],
                                preferred_element_type=jnp.float32) + bn1_ref[...])
    hn = jnp.tanh(jnp.dot(hmid, wn2_ref[...],
                          preferred_element_type=jnp.float32) + bn2_ref[...])
    o_ref[:, :D_HID] = hn
    o_ref[:, D_HID:] = x_ref[...]


def _update(S, H, X2, Wc, bn1, Wn2, bn2):
    return pl.pallas_call(
        _update_body,
        grid=(N_NODES // BM,),
        in_specs=[
            pl.BlockSpec((2, BM, 128), lambda i: (0, i, 0)),
            pl.BlockSpec((BM, D), lambda i: (i, 0)),
            pl.BlockSpec((BM, D_IN), lambda i: (i, 0)),
            pl.BlockSpec((D, D_HID), lambda i: (0, 0)),
            pl.BlockSpec((1, D_HID), lambda i: (0, 0)),
            pl.BlockSpec((D_HID, D_HID), lambda i: (0, 0)),
            pl.BlockSpec((1, D_HID), lambda i: (0, 0)),
        ],
        out_specs=pl.BlockSpec((BM, D), lambda i: (i, 0)),
        out_shape=jax.ShapeDtypeStruct((N_NODES, D), jnp.float32),
    )(S, H, X2, Wc, bn1, Wn2, bn2)


def _sc_gsum_body(go_hbm, gi_hbm, idxo_hbm, idxi_hbm, z_hbm,
                  idxo_t, idxi_t, ubuf, vbuf, zbuf, gsem0, gsem1, wsem):
    c = lax.axis_index("c")
    s = lax.axis_index("s")
    wid = c * 16 + s
    gsems = (gsem0, gsem1)

    pltpu.sync_copy(idxo_hbm.at[wid], idxo_t)
    pltpu.sync_copy(idxi_hbm.at[wid], idxi_t)

    def issue(ci, b):
        pltpu.make_async_copy(go_hbm.at[idxo_t.at[ci]], ubuf.at[b],
                              gsems[b]).start()
        pltpu.make_async_copy(gi_hbm.at[idxi_t.at[ci]], vbuf.at[b],
                              gsems[b]).start()

    def wait(ci, b):
        pltpu.make_async_copy(go_hbm.at[idxo_t.at[ci]], ubuf.at[b],
                              gsems[b]).wait()
        pltpu.make_async_copy(gi_hbm.at[idxi_t.at[ci]], vbuf.at[b],
                              gsems[b]).wait()

    def zdst(ci):
        return z_hbm.at[pl.ds(wid * (NCH_A * C_A) + ci * C_A, C_A)]

    def work(ci, b):
        ub = ubuf.at[b]
        vb = vbuf.at[b]
        zb = zbuf.at[b]

        @pl.when(ci >= 2)
        def _():
            pltpu.make_async_copy(zbuf.at[b], zdst(ci - 2), wsem).wait()

        @pl.loop(0, C_A)
        def _(j):
            for k in range(16):
                sl = pl.ds(16 * k, 16)
                zb[j, sl] = ub[j, sl] + vb[j, sl]

        pltpu.make_async_copy(zbuf.at[b], zdst(ci), wsem).start()

    issue(0, 0)

    @pl.loop(0, NCH_A - 1, step=2)
    def _(ci):
        for b in range(2):
            cur = ci + b
            issue(cur + 1, 1 - b)
            wait(cur, b)
            work(cur, b)

    wait(NCH_A - 1, 0)
    work(NCH_A - 1, 0)
    pltpu.make_async_copy(zbuf.at[1], zdst(NCH_A - 2), wsem).wait()
    pltpu.make_async_copy(zbuf.at[0], zdst(NCH_A - 1), wsem).wait()


def _sc_gsum(Go, Gi, idxo32, idxi32):
    k = pl.kernel(
        _sc_gsum_body,
        out_type=jax.ShapeDtypeStruct((N_EDGES, D_HID), jnp.float32),
        mesh=_mesh,
        compiler_params=_sc_params,
        scratch_types=[
            pltpu.VMEM((NCH_A, C_A), jnp.int32),
            pltpu.VMEM((NCH_A, C_A), jnp.int32),
            pltpu.VMEM((2, C_A, D_HID), jnp.float32),
            pltpu.VMEM((2, C_A, D_HID), jnp.float32),
            pltpu.VMEM((2, C_A, D_HID), jnp.float32),
            pltpu.SemaphoreType.DMA,
            pltpu.SemaphoreType.DMA,
            pltpu.SemaphoreType.DMA,
        ],
    )
    return k(Go, Gi, idxo32, idxi32)


def _sc_msg_body(pa_hbm, pb_hbm, e_hbm, idxo_hbm, idxi_hbm, s_hbm,
                 acc, idxo_t, idxi_t, ubuf, vbuf, ebuf, m12, idx12, zb,
                 gsem0, gsem1, isem0, isem1, isem2, isem3):
    c = lax.axis_index("c")
    s = lax.axis_index("s")
    gsems = (gsem0, gsem1)
    isems = (isem0, isem1, isem2, isem3)

    zv = jnp.zeros((16,), jnp.float32)

    @pl.loop(0, 8)
    def _(r):
        for q in range(8):
            zb[r, pl.ds(16 * q, 16)] = zv

    @pl.when(s < 10)
    def _():
        @pl.loop(0, 125)
        def _(jz):
            pltpu.sync_copy(zb, acc.at[pl.ds(s * 1000 + jz * 8, 8)])

    for p_ in range(2):
        for r_ in list(range(C_B, 48)) + list(range(48 + C_B, 96)):
            for q_ in range(8):
                m12[p_, r_, pl.ds(16 * q_, 16)] = zv

    plsc.subcore_barrier()

    pa = pa_hbm.at[c]
    pb = pb_hbm.at[c]

    def issue_idx(ci, slot):
        pltpu.make_async_copy(idxo_hbm.at[s, ci], idxo_t.at[slot, pl.ds(0, C_B)],
                              isems[slot]).start()
        pltpu.make_async_copy(idxi_hbm.at[s, ci], idxi_t.at[slot, pl.ds(0, C_B)],
                              isems[slot]).start()

    def wait_idx(ci, slot):
        pltpu.make_async_copy(idxo_hbm.at[s, ci], idxo_t.at[slot, pl.ds(0, C_B)],
                              isems[slot]).wait()
        pltpu.make_async_copy(idxi_hbm.at[s, ci], idxi_t.at[slot, pl.ds(0, C_B)],
                              isems[slot]).wait()

    def esrc(ci):
        return e_hbm.at[pl.ds(s * (NCH_B * C_B) + ci * C_B, C_B)]

    def issue(ci, slot, b):
        pltpu.make_async_copy(pa.at[idxo_t.at[slot, pl.ds(0, C_B)]],
                              ubuf.at[b], gsems[b]).start()
        pltpu.make_async_copy(pb.at[idxi_t.at[slot, pl.ds(0, C_B)]],
                              vbuf.at[b], gsems[b]).start()
        pltpu.make_async_copy(esrc(ci), ebuf.at[b], gsems[b]).start()

    def wait(ci, slot, b):
        pltpu.make_async_copy(pa.at[idxo_t.at[slot, pl.ds(0, C_B)]],
                              ubuf.at[b], gsems[b]).wait()
        pltpu.make_async_copy(pb.at[idxi_t.at[slot, pl.ds(0, C_B)]],
                              vbuf.at[b], gsems[b]).wait()
        pltpu.make_async_copy(esrc(ci), ebuf.at[b], gsems[b]).wait()

    lanes8 = lax.iota(jnp.int32, 16) < 8
    zi = jnp.zeros((16,), jnp.int32)

    def work(slot, p):
        ub = ubuf.at[p]
        vb = vbuf.at[p]
        eb = ebuf.at[p]

        for q in range(2):
            idx12[p, pl.ds(16 * q, 16)] = idxi_t[slot, pl.ds(16 * q, 16)]
            idx12[p, pl.ds(48 + 16 * q, 16)] = idxo_t[slot, pl.ds(16 * q, 16)]
        idx12[p, pl.ds(32, 16)] = jnp.where(lanes8,
                                            idxi_t[slot, pl.ds(32, 16)], zi)
        idx12[p, pl.ds(80, 16)] = jnp.where(lanes8,
                                            idxo_t[slot, pl.ds(32, 16)], zi)

        @pl.loop(0, C_B)
        def _(j):
            ev = plsc.load_gather(eb, [jnp.broadcast_to(j, (16,))])
            for k in range(8):
                sl = pl.ds(16 * k, 16)
                m12[p, j, sl] = ev * ub[j, sl]
                m12[p, 48 + j, sl] = ev * vb[j, sl]

        pltpu.sync_copy(m12.at[p], acc.at[idx12.at[p]], add=True)

    def body(ci, b):
        cur = ci + b
        p = b % 2
        slot = b % 4

        @pl.when(cur + 1 < NCH_B)
        def _():
            wait_idx(cur + 1, (slot + 1) % 4)
            issue(cur + 1, (slot + 1) % 4, 1 - p)

        wait(cur, slot, p)
        work(slot, p)

        @pl.when(cur + 3 < NCH_B)
        def _():
            issue_idx(cur + 3, (slot + 3) % 4)

    issue_idx(0, 0)
    issue_idx(1, 1)
    issue_idx(2, 2)
    wait_idx(0, 0)
    issue(0, 0, 0)

    @pl.loop(0, NCH_B - 2, step=4)
    def _(ci):
        for b in range(4):
            body(ci, b)

    body(NCH_B - 2, 0)
    body(NCH_B - 2, 1)

    plsc.subcore_barrier()

    @pl.when(s < 10)
    def _():
        pltpu.sync_copy(acc.at[pl.ds(s * 1000, 1000)],
                        s_hbm.at[c, pl.ds(s * 1000, 1000)])


def _sc_msg(TpA, TpB, e, idxo16, idxi16):
    k = pl.kernel(
        _sc_msg_body,
        out_type=jax.ShapeDtypeStruct((2, N_NODES, 128), jnp.float32),
        mesh=_mesh,
        compiler_params=_sc_params,
        scratch_types=[
            pltpu.VMEM_SHARED((N_NODES, 128), jnp.float32),
            pltpu.VMEM((4, 48), jnp.int32),
            pltpu.VMEM((4, 48), jnp.int32),
            pltpu.VMEM((2, C_B, 128), jnp.float32),
            pltpu.VMEM((2, C_B, 128), jnp.float32),
            pltpu.VMEM((2, C_B), jnp.float32),
            pltpu.VMEM((2, 96, 128), jnp.float32),
            pltpu.VMEM((2, 96), jnp.int32),
            pltpu.VMEM((8, 128), jnp.float32),
            pltpu.SemaphoreType.DMA,
            pltpu.SemaphoreType.DMA,
            pltpu.SemaphoreType.DMA,
            pltpu.SemaphoreType.DMA,
            pltpu.SemaphoreType.DMA,
            pltpu.SemaphoreType.DMA,
        ],
    )
    return k(TpA, TpB, e, idxo16, idxi16)


def kernel(X, edge_in, edge_out, W_in, b_in, We1, be1, We2, be2,
           Wn1, bn1, Wn2, bn2):
    X2 = X[0]
    ii = edge_in[0].astype(jnp.int32)
    oo = edge_out[0].astype(jnp.int32)

    Wc = Wn1[2 * D:]
    b_in2 = b_in.reshape(1, D_HID)
    be1_2 = be1.reshape(1, D_HID)
    be2_2 = be2.reshape(1, 1)
    bn1_2 = bn1.reshape(1, D_HID)
    bn2_2 = bn2.reshape(1, D_HID)

    idxo32 = oo.reshape(32, NCH_A, C_A)
    idxi32 = ii.reshape(32, NCH_A, C_A)
    idxo16 = oo.reshape(16, NCH_B, C_B)
    idxi16 = ii.reshape(16, NCH_B, C_B)

    H = _embed(X2, W_in, b_in2)
    for _ in range(3):
        Go = _proj(H, We1[:D])
        Gi = _proj(H, We1[D:])
        TpA = _proj_split(H, Wn1[:D])
        TpB = _proj_split(H, Wn1[D:2 * D])
        Z = _sc_gsum(Go, Gi, idxo32, idxi32)
        e = _escore(Z, be1_2, We2, be2_2)
        S = _sc_msg(TpA, TpB, e, idxo16, idxi16)
        H = _update(S, H, X2, Wc, bn1_2, Wn2, bn2_2)

    Go = _proj(H, We1[:D])
    Gi = _proj(H, We1[D:])
    Z = _sc_gsum(Go, Gi, idxo32, idxi32)
    e = _escore(Z, be1_2, We2, be2_2)
    return e.reshape(1, N_EDGES)
